# baseline (speedup 1.0000x reference)
import functools

import jax
import jax.numpy as jnp
from jax import lax
from jax.experimental import pallas as pl
from jax.experimental.pallas import tpu as pltpu
from jax.experimental.pallas import tpu_sc as plsc

_NC = 2
_NS = 16
_NW = _NC * _NS

_GCHUNK = 128


def _embed_kernel(B, T, D):
    per_w = B // _NW
    mesh = plsc.VectorSubcoreMesh(core_axis_name="c", subcore_axis_name="s")

    chunks = []
    off = 0
    while off < T:
        n = min(_GCHUNK, T - off)
        chunks.append((off, n))
        off += n

    @functools.partial(
        pl.kernel,
        out_type=jax.ShapeDtypeStruct((B, T, D), jnp.float32),
        mesh=mesh,
        compiler_params=pltpu.CompilerParams(use_tc_tiling_on_sc=False),
        scratch_types=[
            pltpu.VMEM_SHARED((T, D), jnp.float32),
            pltpu.VMEM((T,), jnp.int32),
            pltpu.VMEM((T,), jnp.int32),
            pltpu.VMEM((T, D), jnp.float32),
            pltpu.SemaphoreType.DMA,
        ],
    )
    def k(x_hbm, tab_hbm, pos_hbm, iota_hbm, out_hbm,
          pos_sh, idx_v, iota_v, buf, sem):
        sid = lax.axis_index("s")
        wid = sid * _NC + lax.axis_index("c")
        base = wid * per_w

        @pl.when(sid == 0)
        def _stage_pos():
            pltpu.sync_copy(pos_hbm, pos_sh)

        pltpu.sync_copy(iota_hbm, iota_v)
        plsc.subcore_barrier()

        def one(i, carry):
            b = base + i
            pltpu.sync_copy(x_hbm.at[b], idx_v)
            cps = [
                pltpu.async_copy(
                    tab_hbm.at[idx_v.at[pl.ds(off, n)]],
                    buf.at[pl.ds(off, n)],
                    sem,
                )
                for (off, n) in chunks
            ]
            for cp in cps:
                cp.wait()
            acps = [
                pltpu.async_copy(
                    pos_sh.at[iota_v.at[pl.ds(off, n)]],
                    buf.at[pl.ds(off, n)],
                    sem,
                    add=True,
                )
                for (off, n) in chunks
            ]
            for cp in acps:
                cp.wait()
            pltpu.sync_copy(buf, out_hbm.at[b])
            return carry

        lax.fori_loop(0, per_w, one, 0)

    return k


def kernel(x, token_embed_tab, positional_embeddings):
    B, T = x.shape
    D = token_embed_tab.shape[1]
    iota = jnp.arange(T, dtype=jnp.int32)
    return _embed_kernel(B, T, D)(
        x, token_embed_tab, positional_embeddings, iota)


# speedup vs baseline: 1.1626x; 1.1626x over previous
import functools

import jax
import jax.numpy as jnp
from jax import lax
from jax.experimental import pallas as pl
from jax.experimental.pallas import tpu as pltpu
from jax.experimental.pallas import tpu_sc as plsc

_NC = 2
_NS = 16
_NW = _NC * _NS

_GCHUNK = 128


def _chunks(total):
    out = []
    off = 0
    while off < total:
        n = min(_GCHUNK, total - off)
        out.append((off, n))
        off += n
    return out


def _embed_kernel(N, T, D, step_rows):
    per_w = N // _NW
    n_steps = per_w // step_rows
    assert per_w % step_rows == 0 and n_steps % 2 == 0
    chunks = _chunks(step_rows)
    mesh = plsc.VectorSubcoreMesh(core_axis_name="c", subcore_axis_name="s")

    @functools.partial(
        pl.kernel,
        out_type=jax.ShapeDtypeStruct((N, D), jnp.float32),
        mesh=mesh,
        compiler_params=pltpu.CompilerParams(use_tc_tiling_on_sc=False),
        scratch_types=[
            pltpu.VMEM_SHARED((T, D), jnp.float32),
            pltpu.VMEM((per_w,), jnp.int32),
            pltpu.VMEM((step_rows,), jnp.int32),
            pltpu.VMEM((step_rows, D), jnp.float32),
            pltpu.VMEM((step_rows, D), jnp.float32),
            pltpu.SemaphoreType.DMA,
            pltpu.SemaphoreType.DMA,
            pltpu.SemaphoreType.DMA,
            pltpu.SemaphoreType.DMA,
        ],
    )
    def k(x_hbm, tab_hbm, pos_hbm, iota_hbm, out_hbm,
          pos_sh, idx_v, iota_v, buf0, buf1, sg0, sg1, sa0, sa1):
        sid = lax.axis_index("s")
        wid = sid * _NC + lax.axis_index("c")
        base = wid * per_w

        @pl.when(sid == 0)
        def _stage_pos():
            pltpu.sync_copy(pos_hbm, pos_sh)

        pltpu.sync_copy(x_hbm.at[pl.ds(base, per_w)], idx_v)
        pltpu.sync_copy(iota_hbm, iota_v)
        plsc.subcore_barrier()

        def issue_gather(s, buf, sem):
            o = s * step_rows
            for (off, n) in chunks:
                pltpu.async_copy(
                    tab_hbm.at[idx_v.at[pl.ds(o + off, n)]],
                    buf.at[pl.ds(off, n)],
                    sem,
                )

        def drain(buf, sem):
            pltpu.make_async_copy(
                tab_hbm.at[pl.ds(0, step_rows)], buf, sem).wait()

        def issue_add(buf, sem):
            for (off, n) in chunks:
                pltpu.async_copy(
                    pos_sh.at[iota_v.at[pl.ds(off, n)]],
                    buf.at[pl.ds(off, n)],
                    sem,
                    add=True,
                )

        def finish(s, buf, sem_a):
            issue_add(buf, sem_a)
            drain(buf, sem_a)
            pltpu.sync_copy(buf, out_hbm.at[pl.ds(base + s * step_rows,
                                                  step_rows)])

        issue_gather(0, buf0, sg0)

        def body(s2, carry):
            s0 = 2 * s2
            s1 = s0 + 1
            issue_gather(s1, buf1, sg1)
            drain(buf0, sg0)
            finish(s0, buf0, sa0)

            @pl.when(s0 + 2 < n_steps)
            def _next():
                issue_gather(s0 + 2, buf0, sg0)

            drain(buf1, sg1)
            finish(s1, buf1, sa1)
            return carry

        lax.fori_loop(0, n_steps // 2, body, 0)

    return k


def kernel(x, token_embed_tab, positional_embeddings):
    B, T = x.shape
    D = token_embed_tab.shape[1]
    N = B * T
    step_rows = 2 * T
    iota = jnp.tile(jnp.arange(T, dtype=jnp.int32), step_rows // T)
    out = _embed_kernel(N, T, D, step_rows)(
        x.reshape(N), token_embed_tab, positional_embeddings, iota)
    return out.reshape(B, T, D)
